# Initial kernel scaffold; baseline (speedup 1.0000x reference)
import jax
import jax.numpy as jnp
from jax.experimental import pallas as pl


def kernel(x, y):
    raise NotImplementedError("write your pallas kernel here")


# speedup vs baseline: 1.1532x; 1.1532x over previous
import jax
import jax.numpy as jnp
from jax.experimental import pallas as pl
from jax.experimental.pallas import tpu as pltpu

_ROWS = 2 * 8192
_COLS = 2048
_BLK = 512


def _reduce_kernel(x_ref, y_ref, o_ref, acc_ref):
    i = pl.program_id(0)

    @pl.when(i == 0)
    def _init():
        acc_ref[0] = 0.0
        acc_ref[1] = 0.0

    x = x_ref[...]
    y = y_ref[...]
    d = x - y
    acc_ref[0] += jnp.sum(d * d)
    acc_ref[1] += jnp.sum(y * y)

    @pl.when(i == pl.num_programs(0) - 1)
    def _fin():
        o_ref[0] = acc_ref[0] / acc_ref[1]


def kernel(x, y):
    xf = x.reshape(_ROWS, _COLS)
    yf = y.reshape(_ROWS, _COLS)
    out = pl.pallas_call(
        _reduce_kernel,
        grid=(_ROWS // _BLK,),
        in_specs=[
            pl.BlockSpec((_BLK, _COLS), lambda i: (i, 0)),
            pl.BlockSpec((_BLK, _COLS), lambda i: (i, 0)),
        ],
        out_specs=pl.BlockSpec(memory_space=pltpu.SMEM),
        out_shape=jax.ShapeDtypeStruct((1,), jnp.float32),
        scratch_shapes=[pltpu.SMEM((2,), jnp.float32)],
    )(xf, yf)
    return out[0]


# speedup vs baseline: 1.1709x; 1.0154x over previous
import jax
import jax.numpy as jnp
from jax.experimental import pallas as pl
from jax.experimental.pallas import tpu as pltpu

_ROWS = 2 * 8192
_COLS = 2048
_BLK = 1024


def _reduce_kernel(x_ref, y_ref, o_ref, acc_ref):
    i = pl.program_id(0)

    @pl.when(i == 0)
    def _init():
        acc_ref[0] = 0.0
        acc_ref[1] = 0.0

    x = x_ref[...]
    y = y_ref[...]
    d = x - y
    acc_ref[0] += jnp.sum(d * d)
    acc_ref[1] += jnp.sum(y * y)

    @pl.when(i == pl.num_programs(0) - 1)
    def _fin():
        o_ref[0] = acc_ref[0] / acc_ref[1]


def kernel(x, y):
    xf = x.reshape(_ROWS, _COLS)
    yf = y.reshape(_ROWS, _COLS)
    out = pl.pallas_call(
        _reduce_kernel,
        grid=(_ROWS // _BLK,),
        in_specs=[
            pl.BlockSpec((_BLK, _COLS), lambda i: (i, 0)),
            pl.BlockSpec((_BLK, _COLS), lambda i: (i, 0)),
        ],
        out_specs=pl.BlockSpec(memory_space=pltpu.SMEM),
        out_shape=jax.ShapeDtypeStruct((1,), jnp.float32),
        scratch_shapes=[pltpu.SMEM((2,), jnp.float32)],
    )(xf, yf)
    return out[0]
